# baseline (speedup 1.0000x reference)
import functools

import jax
import jax.numpy as jnp
from jax import lax
from jax.experimental import pallas as pl
from jax.experimental.pallas import tpu as pltpu
from jax.experimental.pallas import tpu_sc as plsc

BATCH = 16384
NUM_PROGRAMS = 100000
DIM = 32
N_FEATS = 16
OUT_DIM = 4 * DIM + N_FEATS

ROWS_PER_SLAB = 4
SLAB_W = ROWS_PER_SLAB * DIM
NUM_SLABS = NUM_PROGRAMS // ROWS_PER_SLAB
TCOLS = 782
TCOLS_PER_SUB = 49
NUM_SLABS_PAD = TCOLS * DIM

NUM_CORES = 2
NUM_SUBCORES = 16
KB = BATCH // NUM_SUBCORES
CHUNK = 32
NCHUNK = KB // CHUNK
LANES = 16

_mesh = plsc.VectorSubcoreMesh(core_axis_name="c", subcore_axis_name="s")


@functools.partial(
    pl.kernel,
    mesh=_mesh,
    out_type=jax.ShapeDtypeStruct((OUT_DIM, BATCH), jnp.float32),
    scratch_types=[
        pltpu.HBM((NUM_SLABS_PAD, SLAB_W), jnp.float32),
        pltpu.HBM((NUM_SLABS_PAD, SLAB_W), jnp.float32),
        [pltpu.VMEM((DIM, SLAB_W), jnp.float32) for _ in range(2)],
        [pltpu.VMEM((DIM, SLAB_W), jnp.float32) for _ in range(2)],
        pltpu.VMEM((DIM, DIM), jnp.float32),
        [pltpu.VMEM((KB,), jnp.int32) for _ in range(2)],
        [pltpu.VMEM((CHUNK,), jnp.int32) for _ in range(4)],
        [pltpu.VMEM((CHUNK, SLAB_W), jnp.float32) for _ in range(4)],
        pltpu.VMEM((N_FEATS, KB), jnp.float32),
        pltpu.VMEM((2 * DIM, KB), jnp.float32),
        [pltpu.SemaphoreType.DMA for _ in range(2)],
        [pltpu.SemaphoreType.DMA for _ in range(2)],
        [pltpu.SemaphoreType.DMA for _ in range(4)],
        pltpu.SemaphoreType.DMA,
        pltpu.SemaphoreType.DMA,
    ],
    compiler_params=pltpu.CompilerParams(
        needs_layout_passes=False, skip_device_barrier=True),
)
def _matchup_sc(idx_hbm, feats_hbm, pw_t, pw_tail_t, tw_t, out_hbm,
                slabs_pw, slabs_tw, binv, boutv, tailv, icols, sids, slabs,
                fv, outv, insems, outsems, gsems, fsem, isem):
    core = lax.axis_index("c")
    sub = lax.axis_index("s")

    def phase1(src, slabs_hbm, has_tail):
        nfull = TCOLS - 1 if has_tail else TCOLS
        j0 = sub * TCOLS_PER_SUB

        def start_in(j, par):
            @pl.when(j < nfull)
            def _():
                off = pl.multiple_of(j * SLAB_W, SLAB_W)
                pltpu.async_copy(
                    src.at[pl.ds(0, DIM), pl.ds(off, SLAB_W)],
                    binv[par], insems[par])

        def transpose_block(src_v, dst_v, n_srows):
            def srow_body(s, _):
                for g in range(SLAB_W // LANES):
                    f0 = (g * LANES) % DIM
                    subr = (g * LANES) // DIM
                    rows = jax.lax.iota(jnp.int32, LANES) + f0
                    cols = jnp.full((LANES,), 0, jnp.int32) + (4 * s + subr)
                    vals = plsc.load_gather(src_v, [rows, cols])
                    dst_v[s, pl.ds(g * LANES, LANES)] = vals
                return ()

            jax.lax.fori_loop(0, n_srows, srow_body, ())

        start_in(j0, 0)
        start_in(j0 + 1, 1)

        def block_step(jl, par):
            j = j0 + jl

            @pl.when(j < nfull)
            def _():
                off = pl.multiple_of(j * SLAB_W, SLAB_W)
                pltpu.make_async_copy(
                    src.at[pl.ds(0, DIM), pl.ds(off, SLAB_W)],
                    binv[par], insems[par]).wait()
                @pl.when(jl >= 2)
                def _():
                    row_p = pl.multiple_of((j - 2) * DIM, DIM)
                    pltpu.make_async_copy(
                        boutv[par], slabs_hbm.at[pl.ds(row_p, DIM)],
                        outsems[par]).wait()

                transpose_block(binv[par], boutv[par], DIM)
                row0 = pl.multiple_of(j * DIM, DIM)
                pltpu.async_copy(
                    boutv[par], slabs_hbm.at[pl.ds(row0, DIM)], outsems[par])

            @pl.when(jl < TCOLS_PER_SUB - 2)
            def _():
                start_in(j + 2, par)

        def pair_body(h, _):
            block_step(2 * h, 0)
            block_step(2 * h + 1, 1)
            return ()

        jax.lax.fori_loop(0, TCOLS_PER_SUB // 2, pair_body, ())
        if TCOLS_PER_SUB % 2:
            block_step(TCOLS_PER_SUB - 1, (TCOLS_PER_SUB - 1) % 2)

        nproc = jnp.clip(nfull - j0, 0, TCOLS_PER_SUB)
        for p in range(2):
            jlp = jnp.where((nproc - 1) % 2 == p, nproc - 1, nproc - 2)

            @pl.when(jlp >= 0)
            def _(jlp=jlp, p=p):
                row_p = pl.multiple_of((j0 + jlp) * DIM, DIM)
                pltpu.make_async_copy(
                    boutv[p], slabs_hbm.at[pl.ds(row_p, DIM)],
                    outsems[p]).wait()

        if has_tail:
            @pl.when(sub == NUM_SUBCORES - 1)
            def _():
                pltpu.sync_copy(pw_tail_t, tailv)
                transpose_block(tailv, boutv[0], 8)
                pltpu.sync_copy(
                    boutv[0].at[pl.ds(0, 8)],
                    slabs_hbm.at[pl.ds((TCOLS - 1) * DIM, 8)])

    def phase2(slabs_hbm, cols, do_feats):
        base = sub * KB
        if do_feats:
            cf = pltpu.async_copy(feats_hbm.at[:, pl.ds(base, KB)], fv, fsem)
        ih = [
            pltpu.async_copy(
                idx_hbm.at[pl.ds(cols[k] * BATCH + base, KB)], icols[k], isem)
            for k in range(2)
        ]
        for h in ih:
            h.wait()

        def fill_and_start(k, buf, ch):
            for i in range(CHUNK // LANES):
                v = icols[k][pl.ds(ch * CHUNK + i * LANES, LANES)]
                sids[buf][pl.ds(i * LANES, LANES)] = (
                    jax.lax.shift_right_logical(v, 2))
            pltpu.async_copy(slabs_hbm.at[sids[buf]], slabs[buf], gsems[buf])

        def extract(k, buf, ch):
            frow = k * DIM
            slab_ref = slabs[buf]
            icol = icols[k]

            def group_body(g, _):
                b0 = ch * CHUNK + g * LANES
                v = icol[pl.ds(b0, LANES)]
                colbase = jax.lax.bitwise_and(v, 3) * DIM
                rows = jax.lax.iota(jnp.int32, LANES) + g * LANES
                for f in range(DIM):
                    vals = plsc.load_gather(slab_ref, [rows, colbase + f])
                    outv[frow + f, pl.ds(b0, LANES)] = vals
                return ()

            jax.lax.fori_loop(0, CHUNK // LANES, group_body, ())

        for k in range(2):
            fill_and_start(k, 2 * k, 0)
            fill_and_start(k, 2 * k + 1, 1)

        def round_par(r, par):
            for k in range(2):
                buf = 2 * k + par
                pltpu.make_async_copy(
                    slabs_hbm.at[sids[buf]], slabs[buf], gsems[buf]).wait()
                extract(k, buf, r)

                @pl.when(r + 2 < NCHUNK)
                def _():
                    fill_and_start(k, buf, r + 2)

        def rounds_body(h, _):
            round_par(2 * h, 0)
            round_par(2 * h + 1, 1)
            return ()

        jax.lax.fori_loop(0, NCHUNK // 2, rounds_body, ())

        pltpu.sync_copy(
            outv.at[pl.ds(0, DIM)],
            out_hbm.at[pl.ds(cols[0] * DIM, DIM), pl.ds(base, KB)])
        pltpu.sync_copy(
            outv.at[pl.ds(DIM, DIM)],
            out_hbm.at[pl.ds(cols[1] * DIM, DIM), pl.ds(base, KB)])
        if do_feats:
            cf.wait()
            pltpu.sync_copy(
                fv, out_hbm.at[pl.ds(4 * DIM, N_FEATS), pl.ds(base, KB)])

    @pl.when(core == 0)
    def _():
        phase1(pw_t, slabs_pw, True)

    @pl.when(core == 1)
    def _():
        phase1(tw_t, slabs_tw, False)

    plsc.subcore_barrier()

    @pl.when(core == 0)
    def _():
        phase2(slabs_pw, (0, 2), True)

    @pl.when(core == 1)
    def _():
        phase2(slabs_tw, (1, 3), False)


def kernel(x, program_weight, team_weight):
    idx_flat = x[:, :4].astype(jnp.int32).T.reshape(-1)
    feats_t = x[:, 4:].T
    pw_tail_t = program_weight[(TCOLS - 1) * SLAB_W:].T
    out_t = _matchup_sc(idx_flat, feats_t, program_weight.T, pw_tail_t,
                        team_weight.T)
    return out_t.T


# speedup vs baseline: 1.3400x; 1.3400x over previous
import functools

import jax
import jax.numpy as jnp
from jax import lax
from jax.experimental import pallas as pl
from jax.experimental.pallas import tpu as pltpu
from jax.experimental.pallas import tpu_sc as plsc

BATCH = 16384
NUM_PROGRAMS = 100000
DIM = 32
N_FEATS = 16
OUT_DIM = 4 * DIM + N_FEATS

ROWS_PER_SLAB = 4
SLAB_W = ROWS_PER_SLAB * DIM
NUM_SLABS = NUM_PROGRAMS // ROWS_PER_SLAB

NUM_CORES = 2
NUM_SUBCORES = 16
NUM_WORKERS = NUM_CORES * NUM_SUBCORES
BPW = BATCH // NUM_WORKERS
CHUNK = 32
NCHUNK = BPW // CHUNK
LANES = 16

_mesh = plsc.VectorSubcoreMesh(core_axis_name="c", subcore_axis_name="s")


@functools.partial(
    pl.kernel,
    mesh=_mesh,
    out_type=jax.ShapeDtypeStruct((OUT_DIM, BATCH), jnp.float32),
    scratch_types=[
        [pltpu.VMEM((BPW,), jnp.int32) for _ in range(4)],
        [pltpu.VMEM((CHUNK,), jnp.int32) for _ in range(8)],
        [pltpu.VMEM((CHUNK, SLAB_W), jnp.float32) for _ in range(8)],
        pltpu.VMEM((N_FEATS, BPW), jnp.float32),
        pltpu.VMEM((4 * DIM, BPW), jnp.float32),
        [pltpu.SemaphoreType.DMA for _ in range(8)],
        pltpu.SemaphoreType.DMA,
        pltpu.SemaphoreType.DMA,
    ],
    compiler_params=pltpu.CompilerParams(
        needs_layout_passes=False, skip_device_barrier=True),
)
def _matchup_sc(idx_hbm, feats_hbm, pws_hbm, tws_hbm, out_hbm,
                icols, sids, slabs, fv, outv, gsems, fsem, isem):
    wid = lax.axis_index("s") * NUM_CORES + lax.axis_index("c")
    base = wid * BPW

    cf = pltpu.async_copy(feats_hbm.at[:, pl.ds(base, BPW)], fv, fsem)
    ih = [
        pltpu.async_copy(
            idx_hbm.at[pl.ds(col * BATCH + base, BPW)], icols[col], isem)
        for col in range(4)
    ]
    for h in ih:
        h.wait()

    tables = [pws_hbm, tws_hbm, pws_hbm, tws_hbm]

    def fill_and_start(col, buf, ch):
        for i in range(CHUNK // LANES):
            v = icols[col][pl.ds(ch * CHUNK + i * LANES, LANES)]
            sids[buf][pl.ds(i * LANES, LANES)] = (
                jax.lax.shift_right_logical(v, 2))
        pltpu.async_copy(tables[col].at[sids[buf]], slabs[buf], gsems[buf])

    def extract(col, buf, ch):
        frow = col * DIM
        slab_ref = slabs[buf]
        icol = icols[col]

        def group_body(g, _):
            b0 = ch * CHUNK + g * LANES
            v = icol[pl.ds(b0, LANES)]
            colbase = jax.lax.bitwise_and(v, 3) * DIM
            rows = jax.lax.iota(jnp.int32, LANES) + g * LANES
            for f in range(DIM):
                vals = plsc.load_gather(slab_ref, [rows, colbase + f])
                outv[frow + f, pl.ds(b0, LANES)] = vals
            return ()

        jax.lax.fori_loop(0, CHUNK // LANES, group_body, ())

    for col in range(4):
        fill_and_start(col, 2 * col, 0)
        fill_and_start(col, 2 * col + 1, 1)

    def round_par(r, par):
        for col in range(4):
            buf = 2 * col + par
            pltpu.make_async_copy(
                tables[col].at[sids[buf]], slabs[buf], gsems[buf]).wait()
            extract(col, buf, r)

            @pl.when(r + 2 < NCHUNK)
            def _():
                fill_and_start(col, buf, r + 2)

    def rounds_body(h, _):
        round_par(2 * h, 0)
        round_par(2 * h + 1, 1)
        return ()

    jax.lax.fori_loop(0, NCHUNK // 2, rounds_body, ())

    pltpu.sync_copy(outv, out_hbm.at[pl.ds(0, 4 * DIM), pl.ds(base, BPW)])
    cf.wait()
    pltpu.sync_copy(fv, out_hbm.at[pl.ds(4 * DIM, N_FEATS), pl.ds(base, BPW)])


def kernel(x, program_weight, team_weight):
    idx_flat = x[:, :4].astype(jnp.int32).T.reshape(-1)
    feats_t = x[:, 4:].T
    pws = program_weight.reshape(NUM_SLABS, SLAB_W)
    tws = team_weight[:NUM_PROGRAMS].reshape(NUM_SLABS, SLAB_W)
    out_t = _matchup_sc(idx_flat, feats_t, pws, tws)
    return out_t.T


# speedup vs baseline: 1.3434x; 1.0026x over previous
import functools

import jax
import jax.numpy as jnp
from jax import lax
from jax.experimental import pallas as pl
from jax.experimental.pallas import tpu as pltpu
from jax.experimental.pallas import tpu_sc as plsc

BATCH = 16384
NUM_PROGRAMS = 100000
DIM = 32
N_FEATS = 16
OUT_DIM = 4 * DIM + N_FEATS

ROWS_PER_SLAB = 4
SLAB_W = ROWS_PER_SLAB * DIM
NUM_SLABS = NUM_PROGRAMS // ROWS_PER_SLAB

NUM_CORES = 2
NUM_SUBCORES = 16
NUM_WORKERS = NUM_CORES * NUM_SUBCORES
BPW = BATCH // NUM_WORKERS
CHUNK = 64
NCHUNK = BPW // CHUNK
LANES = 16

_mesh = plsc.VectorSubcoreMesh(core_axis_name="c", subcore_axis_name="s")


@functools.partial(
    pl.kernel,
    mesh=_mesh,
    out_type=jax.ShapeDtypeStruct((OUT_DIM, BATCH), jnp.float32),
    scratch_types=[
        [pltpu.VMEM((BPW,), jnp.int32) for _ in range(4)],
        [pltpu.VMEM((CHUNK,), jnp.int32) for _ in range(4)],
        [pltpu.VMEM((CHUNK, SLAB_W), jnp.float32) for _ in range(4)],
        pltpu.VMEM((N_FEATS, BPW), jnp.float32),
        pltpu.VMEM((4 * DIM, BPW), jnp.float32),
        [pltpu.SemaphoreType.DMA for _ in range(4)],
        pltpu.SemaphoreType.DMA,
        pltpu.SemaphoreType.DMA,
    ],
    compiler_params=pltpu.CompilerParams(
        needs_layout_passes=False, skip_device_barrier=True),
)
def _matchup_sc(idx_hbm, feats_hbm, pws_hbm, tws_hbm, out_hbm,
                icols, sids, slabs, fv, outv, gsems, fsem, isem):
    wid = lax.axis_index("s") * NUM_CORES + lax.axis_index("c")
    base = wid * BPW

    cf = pltpu.async_copy(feats_hbm.at[:, pl.ds(base, BPW)], fv, fsem)
    ih = [
        pltpu.async_copy(
            idx_hbm.at[pl.ds(col * BATCH + base, BPW)], icols[col], isem)
        for col in range(4)
    ]
    for h in ih:
        h.wait()

    tables = [pws_hbm, tws_hbm, pws_hbm, tws_hbm]

    def fill_and_start(col, ch):
        for i in range(CHUNK // LANES):
            v = icols[col][pl.ds(ch * CHUNK + i * LANES, LANES)]
            sids[col][pl.ds(i * LANES, LANES)] = (
                jax.lax.shift_right_logical(v, 2))
        pltpu.async_copy(tables[col].at[sids[col]], slabs[col], gsems[col])

    def extract(col, ch):
        frow = col * DIM
        slab_ref = slabs[col]
        icol = icols[col]

        def group_body(g, _):
            b0 = ch * CHUNK + g * LANES
            v = icol[pl.ds(b0, LANES)]
            colbase = jax.lax.bitwise_and(v, 3) * DIM
            rows = jax.lax.iota(jnp.int32, LANES) + g * LANES
            for f in range(DIM):
                vals = plsc.load_gather(slab_ref, [rows, colbase + f])
                outv[frow + f, pl.ds(b0, LANES)] = vals
            return ()

        jax.lax.fori_loop(0, CHUNK // LANES, group_body, ())

    for col in range(4):
        fill_and_start(col, 0)

    def round_body(r, _):
        for col in range(4):
            pltpu.make_async_copy(
                tables[col].at[sids[col]], slabs[col], gsems[col]).wait()
            extract(col, r)

            @pl.when(r + 1 < NCHUNK)
            def _():
                fill_and_start(col, r + 1)
        return ()

    jax.lax.fori_loop(0, NCHUNK, round_body, ())

    pltpu.sync_copy(outv, out_hbm.at[pl.ds(0, 4 * DIM), pl.ds(base, BPW)])
    cf.wait()
    pltpu.sync_copy(fv, out_hbm.at[pl.ds(4 * DIM, N_FEATS), pl.ds(base, BPW)])


def kernel(x, program_weight, team_weight):
    idx_flat = x[:, :4].astype(jnp.int32).T.reshape(-1)
    feats_t = x[:, 4:].T
    pws = program_weight.reshape(NUM_SLABS, SLAB_W)
    tws = team_weight[:NUM_PROGRAMS].reshape(NUM_SLABS, SLAB_W)
    out_t = _matchup_sc(idx_flat, feats_t, pws, tws)
    return out_t.T


# speedup vs baseline: 1.3478x; 1.0032x over previous
import functools

import jax
import jax.numpy as jnp
from jax import lax
from jax.experimental import pallas as pl
from jax.experimental.pallas import tpu as pltpu
from jax.experimental.pallas import tpu_sc as plsc

BATCH = 16384
NUM_PROGRAMS = 100000
DIM = 32
N_FEATS = 16
OUT_DIM = 4 * DIM + N_FEATS

ROWS_PER_SLAB = 4
SLAB_W = ROWS_PER_SLAB * DIM
NUM_SLABS = NUM_PROGRAMS // ROWS_PER_SLAB

NUM_CORES = 2
NUM_SUBCORES = 16
NUM_WORKERS = NUM_CORES * NUM_SUBCORES
BPW = BATCH // NUM_WORKERS
CHUNK = 64
NCHUNK = BPW // CHUNK
LANES = 16

_mesh = plsc.VectorSubcoreMesh(core_axis_name="c", subcore_axis_name="s")


@functools.partial(
    pl.kernel,
    mesh=_mesh,
    out_type=jax.ShapeDtypeStruct((OUT_DIM, BATCH), jnp.float32),
    scratch_types=[
        [pltpu.VMEM((BPW,), jnp.int32) for _ in range(4)],
        [pltpu.VMEM((CHUNK,), jnp.int32) for _ in range(4)],
        [pltpu.VMEM((CHUNK, SLAB_W), jnp.float32) for _ in range(4)],
        pltpu.VMEM((4 * DIM, BPW), jnp.float32),
        [pltpu.SemaphoreType.DMA for _ in range(4)],
        pltpu.SemaphoreType.DMA,
        pltpu.SemaphoreType.DMA,
    ],
    compiler_params=pltpu.CompilerParams(
        needs_layout_passes=False, skip_device_barrier=True),
)
def _matchup_sc(idx_hbm, feats_hbm, pws_hbm, tws_hbm, out_hbm,
                icols, sids, slabs, outv, gsems, fsem, isem):
    wid = lax.axis_index("s") * NUM_CORES + lax.axis_index("c")
    base = wid * BPW

    cf = pltpu.async_copy(
        feats_hbm.at[:, pl.ds(base, BPW)],
        out_hbm.at[pl.ds(4 * DIM, N_FEATS), pl.ds(base, BPW)], fsem)
    ih = [
        pltpu.async_copy(
            idx_hbm.at[pl.ds(col * BATCH + base, BPW)], icols[col], isem)
        for col in range(4)
    ]
    for h in ih:
        h.wait()

    tables = [pws_hbm, tws_hbm, pws_hbm, tws_hbm]

    def fill_and_start(col, ch):
        for i in range(CHUNK // LANES):
            v = icols[col][pl.ds(ch * CHUNK + i * LANES, LANES)]
            sids[col][pl.ds(i * LANES, LANES)] = (
                jax.lax.shift_right_logical(v, 2))
        pltpu.async_copy(tables[col].at[sids[col]], slabs[col], gsems[col])

    def extract(col, ch):
        frow = col * DIM
        slab_ref = slabs[col]
        icol = icols[col]

        def group_body(g, _):
            b0 = ch * CHUNK + g * LANES
            v = icol[pl.ds(b0, LANES)]
            colbase = jax.lax.bitwise_and(v, 3) * DIM
            rows = jax.lax.iota(jnp.int32, LANES) + g * LANES
            for f in range(DIM):
                vals = plsc.load_gather(slab_ref, [rows, colbase + f])
                outv[frow + f, pl.ds(b0, LANES)] = vals
            return ()

        jax.lax.fori_loop(0, CHUNK // LANES, group_body, ())

    for col in range(4):
        fill_and_start(col, 0)

    def round_body(r, _):
        for col in range(4):
            pltpu.make_async_copy(
                tables[col].at[sids[col]], slabs[col], gsems[col]).wait()
            extract(col, r)

            @pl.when(r + 1 < NCHUNK)
            def _():
                fill_and_start(col, r + 1)
        return ()

    jax.lax.fori_loop(0, NCHUNK, round_body, ())

    pltpu.sync_copy(outv, out_hbm.at[pl.ds(0, 4 * DIM), pl.ds(base, BPW)])
    cf.wait()


def kernel(x, program_weight, team_weight):
    idx_flat = x[:, :4].astype(jnp.int32).T.reshape(-1)
    feats_t = x[:, 4:].T
    pws = program_weight.reshape(NUM_SLABS, SLAB_W)
    tws = team_weight[:NUM_PROGRAMS].reshape(NUM_SLABS, SLAB_W)
    out_t = _matchup_sc(idx_flat, feats_t, pws, tws)
    return out_t.T
